# baseline (device time: 205947 ns/iter reference)
import jax
import jax.numpy as jnp
from jax import lax
from jax.experimental import pallas as pl
from jax.experimental.pallas import tpu as pltpu

N_DEV = 8
N_RING = 4
DEPTH = 3
N_HOPS = 2 * (N_DEV - 1)


def _gelu(y):
    c = 0.7978845608028654
    return 0.5 * y * (1.0 + jnp.tanh(c * (y + 0.044715 * y * y * y)))


def kernel(x, w_mat):
    m, k_sh = x.shape
    _, n = w_mat.shape
    ch = m // N_DEV
    nq = n // N_RING

    def body(x_ref, w_ref, out_ref, comm, xbf, wbf, sems):
        my = lax.axis_index("i")
        left = lax.rem(my + N_DEV - 1, N_DEV)
        right = lax.rem(my + 1, N_DEV)

        barrier_sem = pltpu.get_barrier_semaphore()
        for nbr in (left, right):
            pl.semaphore_signal(
                barrier_sem, inc=1,
                device_id=(nbr,), device_id_type=pl.DeviceIdType.MESH,
            )
        pl.semaphore_wait(barrier_sem, 2)

        wbf[...] = w_ref[...].astype(jnp.bfloat16)
        own_rows = pl.ds(my * ch, ch)
        xbf[own_rows, :] = x_ref[own_rows, :].astype(jnp.bfloat16)

        def rdir(q):
            return 0 if q < 2 else 1

        dsts = (right, right, left, left)
        cols = tuple(slice(q * nq, (q + 1) * nq) for q in range(N_RING))

        def partial_q(c, q):
            xs = xbf[pl.ds(c * ch, ch), :]
            return lax.dot_general(
                xs, wbf[:, cols[q]],
                (((1,), (0,)), ((), ())),
                preferred_element_type=jnp.float32,
            )

        descs = {}

        def sem_pair(s, q):
            return (sems.at[q, 0, s % DEPTH], sems.at[q, 1, (s + 1) % DEPTH])

        def make_rs(s, q):
            send_sem, recv_sem = sem_pair(s, q)
            return pltpu.make_async_remote_copy(
                src_ref=comm.at[q, s % DEPTH],
                dst_ref=comm.at[q, (s + 1) % DEPTH],
                send_sem=send_sem,
                recv_sem=recv_sem,
                device_id=(dsts[q],),
                device_id_type=pl.DeviceIdType.MESH,
            )

        def make_ag(s, q, c_send):
            ref = out_ref.at[pl.ds(c_send * ch, ch), cols[q]]
            send_sem, recv_sem = sem_pair(s, q)
            return pltpu.make_async_remote_copy(
                src_ref=ref,
                dst_ref=ref,
                send_sem=send_sem,
                recv_sem=recv_sem,
                device_id=(dsts[q],),
                device_id_type=pl.DeviceIdType.MESH,
            )

        def start(s, q, desc):
            pred_q = q ^ 1
            pred_s = s - 1 if q % 2 == 0 else s
            if (pred_s, pred_q) in descs:
                descs[(pred_s, pred_q)].wait_send()
            descs[(s, q)] = desc
            desc.start()

        for q in (0, 2):
            comm[q, 0] = partial_q(my, q).astype(jnp.bfloat16)
            start(0, q, make_rs(0, q))
        for q in (1, 3):
            comm[q, 0] = partial_q(my, q).astype(jnp.bfloat16)
        xbf[...] = x_ref[...].astype(jnp.bfloat16)
        for q in (1, 3):
            start(0, q, make_rs(0, q))

        for s in range(N_DEV - 1):
            recv_slot = (s + 1) % DEPTH
            c_rs = (
                lax.rem(my + 2 * N_DEV - s - 1, N_DEV),
                lax.rem(my + s + 1, N_DEV),
            )
            for q in (0, 2, 1, 3):
                descs[(s, q)].wait_recv()
                comm[q, recv_slot] = (
                    comm[q, recv_slot].astype(jnp.float32)
                    + partial_q(c_rs[rdir(q)], q)
                ).astype(jnp.bfloat16)
                if s < N_DEV - 2:
                    start(s + 1, q, make_rs(s + 1, q))

        red_slot = (N_DEV - 1) % DEPTH
        own = (lax.rem(my + 1, N_DEV), lax.rem(my + N_DEV - 1, N_DEV))
        for q in (0, 2, 1, 3):
            ge = _gelu(comm[q, red_slot].astype(jnp.float32))
            out_ref[pl.ds(own[rdir(q)] * ch, ch), cols[q]] = ge.astype(
                jnp.bfloat16
            )
            start(N_DEV - 1, q, make_ag(N_DEV - 1, q, own[rdir(q)]))

        for t in range(N_DEV - 1):
            s = N_DEV - 1 + t
            c_ag = (
                lax.rem(my + 2 * N_DEV - t, N_DEV),
                lax.rem(my + t, N_DEV),
            )
            for q in (0, 2, 1, 3):
                descs[(s, q)].wait_recv()
                if t < N_DEV - 2:
                    start(s + 1, q, make_ag(s + 1, q, c_ag[rdir(q)]))

        for q in (1, 3):
            descs[(N_HOPS - 1, q)].wait_send()

    out_shape = jax.ShapeDtypeStruct((m, n), jnp.bfloat16)
    return pl.pallas_call(
        body,
        out_shape=out_shape,
        in_specs=[
            pl.BlockSpec(memory_space=pltpu.VMEM),
            pl.BlockSpec(memory_space=pltpu.VMEM),
        ],
        out_specs=pl.BlockSpec(memory_space=pltpu.VMEM),
        scratch_shapes=[
            pltpu.VMEM((N_RING, DEPTH, ch, nq), jnp.bfloat16),
            pltpu.VMEM((m, k_sh), jnp.bfloat16),
            pltpu.VMEM((k_sh, n), jnp.bfloat16),
            pltpu.SemaphoreType.DMA((N_RING, 2, DEPTH)),
        ],
        compiler_params=pltpu.CompilerParams(
            collective_id=0,
            vmem_limit_bytes=60 * 1024 * 1024,
        ),
    )(x, w_mat)


# device time: 187832 ns/iter; 1.0964x vs baseline; 1.0964x over previous
import jax
import jax.numpy as jnp
from jax import lax
from jax.experimental import pallas as pl
from jax.experimental.pallas import tpu as pltpu

N_DEV = 8
N_RING = 4
DEPTH = 3
N_HOPS = 2 * (N_DEV - 1)


def _gelu(y):
    c = 0.7978845608028654
    return 0.5 * y * (1.0 + jnp.tanh(c * (y + 0.044715 * y * y * y)))


def kernel(x, w_mat):
    m, k_sh = x.shape
    _, n = w_mat.shape
    ch = m // N_DEV
    nq = n // N_RING

    def body(x_ref, w_ref, out_ref, comm, gather, xbf, wbf, sems, osems):
        my = lax.axis_index("i")
        left = lax.rem(my + N_DEV - 1, N_DEV)
        right = lax.rem(my + 1, N_DEV)

        barrier_sem = pltpu.get_barrier_semaphore()
        for nbr in (left, right):
            pl.semaphore_signal(
                barrier_sem, inc=1,
                device_id=(nbr,), device_id_type=pl.DeviceIdType.MESH,
            )
        pl.semaphore_wait(barrier_sem, 2)

        wbf[...] = w_ref[...].astype(jnp.bfloat16)
        own_rows = pl.ds(my * ch, ch)
        xbf[own_rows, :] = x_ref[own_rows, :].astype(jnp.bfloat16)

        def rdir(q):
            return 0 if q < 2 else 1

        dsts = (right, right, left, left)
        cols = tuple(slice(q * nq, (q + 1) * nq) for q in range(N_RING))

        def partial_q(c, q):
            xs = xbf[pl.ds(c * ch, ch), :]
            return lax.dot_general(
                xs, wbf[:, cols[q]],
                (((1,), (0,)), ((), ())),
                preferred_element_type=jnp.float32,
            )

        descs = {}

        def sem_pair(s, q):
            return (sems.at[q, 0, s % DEPTH], sems.at[q, 1, (s + 1) % DEPTH])

        def make_rs(s, q):
            send_sem, recv_sem = sem_pair(s, q)
            return pltpu.make_async_remote_copy(
                src_ref=comm.at[q, s % DEPTH],
                dst_ref=comm.at[q, (s + 1) % DEPTH],
                send_sem=send_sem,
                recv_sem=recv_sem,
                device_id=(dsts[q],),
                device_id_type=pl.DeviceIdType.MESH,
            )

        def make_ag(s, q, c_send):
            ref = gather.at[pl.ds(c_send * ch, ch), cols[q]]
            send_sem, recv_sem = sem_pair(s, q)
            return pltpu.make_async_remote_copy(
                src_ref=ref,
                dst_ref=ref,
                send_sem=send_sem,
                recv_sem=recv_sem,
                device_id=(dsts[q],),
                device_id_type=pl.DeviceIdType.MESH,
            )

        def start(s, q, desc):
            if s >= DEPTH:
                descs[(s - DEPTH, q)].wait_send()
            descs[(s, q)] = desc
            desc.start()

        for q in (0, 2):
            comm[q, 0] = partial_q(my, q).astype(jnp.bfloat16)
            start(0, q, make_rs(0, q))
        for q in (1, 3):
            comm[q, 0] = partial_q(my, q).astype(jnp.bfloat16)
        xbf[...] = x_ref[...].astype(jnp.bfloat16)
        for q in (1, 3):
            start(0, q, make_rs(0, q))

        for s in range(N_DEV - 1):
            recv_slot = (s + 1) % DEPTH
            c_rs = (
                lax.rem(my + 2 * N_DEV - s - 1, N_DEV),
                lax.rem(my + s + 1, N_DEV),
            )
            for q in (0, 2, 1, 3):
                descs[(s, q)].wait_recv()
                comm[q, recv_slot] = (
                    comm[q, recv_slot].astype(jnp.float32)
                    + partial_q(c_rs[rdir(q)], q)
                ).astype(jnp.bfloat16)
                if s < N_DEV - 2:
                    start(s + 1, q, make_rs(s + 1, q))

        ocopies = {}

        def stream_out(idx, q, c):
            if idx >= 2:
                ocopies[(idx - 2, q)].wait()
            rows = pl.ds(c * ch, ch)
            cp = pltpu.make_async_copy(
                gather.at[rows, cols[q]],
                out_ref.at[rows, cols[q]],
                osems.at[q, idx % 2],
            )
            ocopies[(idx, q)] = cp
            cp.start()

        red_slot = (N_DEV - 1) % DEPTH
        own = (lax.rem(my + 1, N_DEV), lax.rem(my + N_DEV - 1, N_DEV))
        for q in (0, 2, 1, 3):
            ge = _gelu(comm[q, red_slot].astype(jnp.float32))
            gather[pl.ds(own[rdir(q)] * ch, ch), cols[q]] = ge.astype(
                jnp.bfloat16
            )
            start(N_DEV - 1, q, make_ag(N_DEV - 1, q, own[rdir(q)]))
            stream_out(0, q, own[rdir(q)])

        for t in range(N_DEV - 1):
            s = N_DEV - 1 + t
            c_ag = (
                lax.rem(my + 2 * N_DEV - t, N_DEV),
                lax.rem(my + t, N_DEV),
            )
            for q in (0, 2, 1, 3):
                descs[(s, q)].wait_recv()
                if t < N_DEV - 2:
                    start(s + 1, q, make_ag(s + 1, q, c_ag[rdir(q)]))
                stream_out(t + 1, q, c_ag[rdir(q)])

        for s in range(N_HOPS - DEPTH, N_HOPS):
            for q in range(N_RING):
                descs[(s, q)].wait_send()
        for idx in (N_DEV - 2, N_DEV - 1):
            for q in range(N_RING):
                ocopies[(idx, q)].wait()

    out_shape = jax.ShapeDtypeStruct((m, n), jnp.bfloat16)
    return pl.pallas_call(
        body,
        out_shape=out_shape,
        in_specs=[
            pl.BlockSpec(memory_space=pltpu.VMEM),
            pl.BlockSpec(memory_space=pltpu.VMEM),
        ],
        out_specs=pl.BlockSpec(memory_space=pl.ANY),
        scratch_shapes=[
            pltpu.VMEM((N_RING, DEPTH, ch, nq), jnp.bfloat16),
            pltpu.VMEM((m, n), jnp.bfloat16),
            pltpu.VMEM((m, k_sh), jnp.bfloat16),
            pltpu.VMEM((k_sh, n), jnp.bfloat16),
            pltpu.SemaphoreType.DMA((N_RING, 2, DEPTH)),
            pltpu.SemaphoreType.DMA((N_RING, 2)),
        ],
        compiler_params=pltpu.CompilerParams(
            collective_id=0,
            vmem_limit_bytes=60 * 1024 * 1024,
        ),
    )(x, w_mat)


# device time: 186607 ns/iter; 1.1036x vs baseline; 1.0066x over previous
import jax
import jax.numpy as jnp
from jax import lax
from jax.experimental import pallas as pl
from jax.experimental.pallas import tpu as pltpu

N_DEV = 8
N_RING = 4
DEPTH = 3
N_HOPS = 2 * (N_DEV - 1)


def _gelu(y):
    c = 0.7978845608028654
    return 0.5 * y * (1.0 + jnp.tanh(c * (y + 0.044715 * y * y * y)))


def kernel(x, w_mat):
    m, k_sh = x.shape
    _, n = w_mat.shape
    ch = m // N_DEV
    nq = n // N_RING

    def body(x_ref, w_ref, out_ref, comm, gather, xbf, wbf, sems, osems):
        my = lax.axis_index("i")
        def ring_id(pos):
            return jnp.where(pos < 4, pos, 11 - pos)

        r = ring_id(my)
        right = ring_id(lax.rem(r + 1, N_DEV))
        left = ring_id(lax.rem(r + N_DEV - 1, N_DEV))

        barrier_sem = pltpu.get_barrier_semaphore()
        for nbr in (left, right):
            pl.semaphore_signal(
                barrier_sem, inc=1,
                device_id=(nbr,), device_id_type=pl.DeviceIdType.MESH,
            )
        pl.semaphore_wait(barrier_sem, 2)

        wbf[...] = w_ref[...].astype(jnp.bfloat16)
        own_rows = pl.ds(r * ch, ch)
        xbf[own_rows, :] = x_ref[own_rows, :].astype(jnp.bfloat16)

        def rdir(q):
            return 0 if q < 2 else 1

        dsts = (right, right, left, left)
        cols = tuple(slice(q * nq, (q + 1) * nq) for q in range(N_RING))

        def partial_q(c, q):
            xs = xbf[pl.ds(c * ch, ch), :]
            return lax.dot_general(
                xs, wbf[:, cols[q]],
                (((1,), (0,)), ((), ())),
                preferred_element_type=jnp.float32,
            )

        descs = {}

        def sem_pair(s, q):
            return (sems.at[q, 0, s % DEPTH], sems.at[q, 1, (s + 1) % DEPTH])

        def make_rs(s, q):
            send_sem, recv_sem = sem_pair(s, q)
            return pltpu.make_async_remote_copy(
                src_ref=comm.at[q, s % DEPTH],
                dst_ref=comm.at[q, (s + 1) % DEPTH],
                send_sem=send_sem,
                recv_sem=recv_sem,
                device_id=(dsts[q],),
                device_id_type=pl.DeviceIdType.MESH,
            )

        def make_ag(s, q, c_send):
            ref = gather.at[pl.ds(c_send * ch, ch), cols[q]]
            send_sem, recv_sem = sem_pair(s, q)
            return pltpu.make_async_remote_copy(
                src_ref=ref,
                dst_ref=ref,
                send_sem=send_sem,
                recv_sem=recv_sem,
                device_id=(dsts[q],),
                device_id_type=pl.DeviceIdType.MESH,
            )

        def start(s, q, desc):
            if s >= DEPTH:
                descs[(s - DEPTH, q)].wait_send()
            descs[(s, q)] = desc
            desc.start()

        for q in (0, 2):
            comm[q, 0] = partial_q(r, q).astype(jnp.bfloat16)
            start(0, q, make_rs(0, q))
        for q in (1, 3):
            comm[q, 0] = partial_q(r, q).astype(jnp.bfloat16)
        xbf[...] = x_ref[...].astype(jnp.bfloat16)
        for q in (1, 3):
            start(0, q, make_rs(0, q))

        for s in range(N_DEV - 1):
            recv_slot = (s + 1) % DEPTH
            c_rs = (
                lax.rem(r + 2 * N_DEV - s - 1, N_DEV),
                lax.rem(r + s + 1, N_DEV),
            )
            for q in (0, 2, 1, 3):
                descs[(s, q)].wait_recv()
                comm[q, recv_slot] = (
                    comm[q, recv_slot].astype(jnp.float32)
                    + partial_q(c_rs[rdir(q)], q)
                ).astype(jnp.bfloat16)
                if s < N_DEV - 2:
                    start(s + 1, q, make_rs(s + 1, q))

        ocopies = {}

        def stream_out(idx, q, c):
            if idx >= 2:
                ocopies[(idx - 2, q)].wait()
            rows = pl.ds(c * ch, ch)
            cp = pltpu.make_async_copy(
                gather.at[rows, cols[q]],
                out_ref.at[rows, cols[q]],
                osems.at[q, idx % 2],
            )
            ocopies[(idx, q)] = cp
            cp.start()

        red_slot = (N_DEV - 1) % DEPTH
        own = (lax.rem(r + 1, N_DEV), lax.rem(r + N_DEV - 1, N_DEV))
        for q in (0, 2, 1, 3):
            ge = _gelu(comm[q, red_slot].astype(jnp.float32))
            gather[pl.ds(own[rdir(q)] * ch, ch), cols[q]] = ge.astype(
                jnp.bfloat16
            )
            start(N_DEV - 1, q, make_ag(N_DEV - 1, q, own[rdir(q)]))
            stream_out(0, q, own[rdir(q)])

        for t in range(N_DEV - 1):
            s = N_DEV - 1 + t
            c_ag = (
                lax.rem(r + 2 * N_DEV - t, N_DEV),
                lax.rem(r + t, N_DEV),
            )
            for q in (0, 2, 1, 3):
                descs[(s, q)].wait_recv()
                if t < N_DEV - 2:
                    start(s + 1, q, make_ag(s + 1, q, c_ag[rdir(q)]))
                stream_out(t + 1, q, c_ag[rdir(q)])

        for s in range(N_HOPS - DEPTH, N_HOPS):
            for q in range(N_RING):
                descs[(s, q)].wait_send()
        for idx in (N_DEV - 2, N_DEV - 1):
            for q in range(N_RING):
                ocopies[(idx, q)].wait()

    out_shape = jax.ShapeDtypeStruct((m, n), jnp.bfloat16)
    return pl.pallas_call(
        body,
        out_shape=out_shape,
        in_specs=[
            pl.BlockSpec(memory_space=pltpu.VMEM),
            pl.BlockSpec(memory_space=pltpu.VMEM),
        ],
        out_specs=pl.BlockSpec(memory_space=pl.ANY),
        scratch_shapes=[
            pltpu.VMEM((N_RING, DEPTH, ch, nq), jnp.bfloat16),
            pltpu.VMEM((m, n), jnp.bfloat16),
            pltpu.VMEM((m, k_sh), jnp.bfloat16),
            pltpu.VMEM((k_sh, n), jnp.bfloat16),
            pltpu.SemaphoreType.DMA((N_RING, 2, DEPTH)),
            pltpu.SemaphoreType.DMA((N_RING, 2)),
        ],
        compiler_params=pltpu.CompilerParams(
            collective_id=0,
            vmem_limit_bytes=60 * 1024 * 1024,
        ),
    )(x, w_mat)
